# baseline (speedup 1.0000x reference)
import jax
import jax.numpy as jnp
from jax import lax
from jax.experimental import pallas as pl
from jax.experimental.pallas import tpu as pltpu
from jax.experimental.pallas import tpu_sc as plsc

NC = 2
NS = 16
NW = NC * NS
L = 16

N = 2097152
S = 2
NP = N // S
P = NP // NW
C = 4096
NCHUNKS = P // C


def _rsqrt(x):
    i = lax.bitcast_convert_type(x, jnp.int32)
    i = jnp.int32(0x5F3759DF) - lax.shift_right_logical(i, 1)
    y = lax.bitcast_convert_type(i, jnp.float32)
    for _ in range(3):
        y = y * (1.5 - 0.5 * x * y * y)
    return y


def _sc_body(x_hbm, y_hbm, z_hbm,
             ox_hbm, oy_hbm, oz_hbm, nx_hbm, ny_hbm, nz_hbm, mk_hbm,
             *sbuf):
    c = lax.axis_index("c")
    s = lax.axis_index("s")
    wid = s * NC + c
    base_p = wid * P

    in_hbm = (x_hbm, y_hbm, z_hbm)
    out_hbm = (ox_hbm, oy_hbm, oz_hbm, nx_hbm, ny_hbm, nz_hbm, mk_hbm)
    inb = (sbuf[0:3], sbuf[10:13])
    outb = (sbuf[3:10], sbuf[13:20])
    in_s = sbuf[20:22]
    out_s = sbuf[22:24]

    def start_in(k, b):
        off = base_p + k * C
        for h, v in zip(in_hbm, inb[b]):
            pltpu.async_copy(h.at[pl.ds(off, C)], v, in_s[b])

    def wait_in(k, b):
        off = base_p + k * C
        for h, v in zip(in_hbm, inb[b]):
            pltpu.make_async_copy(h.at[pl.ds(off, C)], v, in_s[b]).wait()

    def start_out(k, b):
        off = base_p + k * C
        for v, h in zip(outb[b], out_hbm):
            pltpu.async_copy(v, h.at[pl.ds(off, C)], out_s[b])

    def drain_out(k, b):
        off = base_p + k * C
        for v, h in zip(outb[b], out_hbm):
            pltpu.make_async_copy(v, h.at[pl.ds(off, C)], out_s[b]).wait()

    start_in(0, 0)

    def do_pair(k2, carry):
        for b in range(2):
            k = k2 * 2 + b

            @pl.when(k + 1 < NCHUNKS)
            def _():
                start_in(k + 1, 1 - b)

            wait_in(k, b)

            @pl.when(k >= 2)
            def _():
                drain_out(k - 2, b)

            x_v, y_v, z_v = inb[b]
            ox_v, oy_v, oz_v, nx_v, ny_v, nz_v, mk_v = outb[b]

            @plsc.parallel_loop(0, C // L, step=1, unroll=8)
            def do_group(g):
                gb = g * L
                x = x_v[pl.ds(gb, L)]
                y = y_v[pl.ds(gb, L)]
                z = z_v[pl.ds(gb, L)]
                r2 = x * x + y * y + z * z
                inv_r = _rsqrt(r2)
                sc = jnp.minimum(jnp.float32(1.0), 2.0 * inv_r - 1.0)
                ox_v[pl.ds(gb, L)] = x * sc
                oy_v[pl.ds(gb, L)] = y * sc
                oz_v[pl.ds(gb, L)] = z * sc
                nx_v[pl.ds(gb, L)] = x * inv_r
                ny_v[pl.ds(gb, L)] = y * inv_r
                nz_v[pl.ds(gb, L)] = z * inv_r
                mk_v[pl.ds(gb, L)] = (r2 > 1.0).astype(jnp.int32)

            start_out(k, b)
        return carry

    lax.fori_loop(0, NCHUNKS // 2, do_pair, 0)
    drain_out(NCHUNKS - 2, 0)
    drain_out(NCHUNKS - 1, 1)


def _run(x, y, z):
    mesh = plsc.VectorSubcoreMesh(core_axis_name="c", subcore_axis_name="s")
    f = jax.ShapeDtypeStruct((NP,), jnp.float32)
    return pl.kernel(
        _sc_body,
        out_type=[f, f, f, f, f, f, jax.ShapeDtypeStruct((NP,), jnp.int32)],
        mesh=mesh,
        compiler_params=pltpu.CompilerParams(
            needs_layout_passes=False, use_tc_tiling_on_sc=False
        ),
        scratch_types=(
            [pltpu.VMEM((C,), jnp.float32)] * 9
            + [pltpu.VMEM((C,), jnp.int32)]
        ) * 2
        + [pltpu.SemaphoreType.DMA] * 4,
    )(x, y, z)


@jax.jit
def _full(xt):
    outs, nbs, mks = [], [], []
    for i in range(S):
        rows = xt[i * NP:(i + 1) * NP]
        ox, oy, oz, nx, ny, nz, mk = _run(rows[:, 0], rows[:, 1], rows[:, 2])
        outs.append(jnp.stack([ox, oy, oz], axis=1))
        nbs.append(jnp.stack([nx, ny, nz], axis=1))
        mks.append(mk.astype(bool))
    return (
        jnp.concatenate(outs, axis=0),
        jnp.concatenate(nbs, axis=0),
        jnp.concatenate(mks, axis=0),
    )


def kernel(xt):
    return _full(xt)


# speedup vs baseline: 1.6833x; 1.6833x over previous
import jax
import jax.numpy as jnp
from jax import lax
from jax.experimental import pallas as pl
from jax.experimental.pallas import tpu as pltpu
from jax.experimental.pallas import tpu_sc as plsc

NC = 2
NS = 16
NW = NC * NS
L = 16

N = 2097152
G = N // 128
RW = G // NW
CR = 32
NCHUNKS = RW // CR


def _rsqrt(x):
    i = lax.bitcast_convert_type(x, jnp.int32)
    i = jnp.int32(0x5F3759DF) - lax.shift_right_logical(i, 1)
    y = lax.bitcast_convert_type(i, jnp.float32)
    for _ in range(3):
        y = y * (1.5 - 0.5 * x * y * y)
    return y


def _sc_body(x_hbm, y_hbm, z_hbm,
             ox_hbm, oy_hbm, oz_hbm, nx_hbm, ny_hbm, nz_hbm, mk_hbm,
             *sbuf):
    c = lax.axis_index("c")
    s = lax.axis_index("s")
    wid = s * NC + c
    base_r = wid * RW

    in_hbm = (x_hbm, y_hbm, z_hbm)
    out_hbm = (ox_hbm, oy_hbm, oz_hbm, nx_hbm, ny_hbm, nz_hbm, mk_hbm)
    inb = (sbuf[0:3], sbuf[10:13])
    outb = (sbuf[3:10], sbuf[13:20])
    in_s = sbuf[20:22]
    out_s = sbuf[22:24]

    def start_in(k, b):
        off = base_r + k * CR
        for h, v in zip(in_hbm, inb[b]):
            pltpu.async_copy(h.at[pl.ds(off, CR)], v, in_s[b])

    def wait_in(k, b):
        off = base_r + k * CR
        for h, v in zip(in_hbm, inb[b]):
            pltpu.make_async_copy(h.at[pl.ds(off, CR)], v, in_s[b]).wait()

    def start_out(k, b):
        off = base_r + k * CR
        for v, h in zip(outb[b], out_hbm):
            pltpu.async_copy(v, h.at[pl.ds(off, CR)], out_s[b])

    def drain_out(k, b):
        off = base_r + k * CR
        for v, h in zip(outb[b], out_hbm):
            pltpu.make_async_copy(v, h.at[pl.ds(off, CR)], out_s[b]).wait()

    start_in(0, 0)

    def do_pair(k2, carry):
        for b in range(2):
            k = k2 * 2 + b

            @pl.when(k + 1 < NCHUNKS)
            def _():
                start_in(k + 1, 1 - b)

            wait_in(k, b)

            @pl.when(k >= 2)
            def _():
                drain_out(k - 2, b)

            x_v, y_v, z_v = inb[b]
            ox_v, oy_v, oz_v, nx_v, ny_v, nz_v, mk_v = outb[b]

            @plsc.parallel_loop(0, CR * 8, step=1, unroll=8)
            def do_group(g):
                r = g // 8
                col = (g % 8) * L
                x = x_v[r, pl.ds(col, L)]
                y = y_v[r, pl.ds(col, L)]
                z = z_v[r, pl.ds(col, L)]
                r2 = x * x + y * y + z * z
                inv_r = _rsqrt(r2)
                sc = jnp.minimum(jnp.float32(1.0), 2.0 * inv_r - 1.0)
                ox_v[r, pl.ds(col, L)] = x * sc
                oy_v[r, pl.ds(col, L)] = y * sc
                oz_v[r, pl.ds(col, L)] = z * sc
                nx_v[r, pl.ds(col, L)] = x * inv_r
                ny_v[r, pl.ds(col, L)] = y * inv_r
                nz_v[r, pl.ds(col, L)] = z * inv_r
                mk_v[r, pl.ds(col, L)] = (r2 > 1.0).astype(jnp.int32)

            start_out(k, b)
        return carry

    lax.fori_loop(0, NCHUNKS // 2, do_pair, 0)
    drain_out(NCHUNKS - 2, 0)
    drain_out(NCHUNKS - 1, 1)


def _run(x, y, z):
    mesh = plsc.VectorSubcoreMesh(core_axis_name="c", subcore_axis_name="s")
    f = jax.ShapeDtypeStruct((G, 128), jnp.float32)
    return pl.kernel(
        _sc_body,
        out_type=[f, f, f, f, f, f, jax.ShapeDtypeStruct((G, 128), jnp.int32)],
        mesh=mesh,
        compiler_params=pltpu.CompilerParams(
            needs_layout_passes=False, use_tc_tiling_on_sc=False
        ),
        scratch_types=(
            [pltpu.VMEM((CR, 128), jnp.float32)] * 9
            + [pltpu.VMEM((CR, 128), jnp.int32)]
        ) * 2
        + [pltpu.SemaphoreType.DMA] * 4,
    )(x, y, z)


@jax.jit
def _full(xt):
    x2 = xt[:, 0].reshape(G, 128)
    y2 = xt[:, 1].reshape(G, 128)
    z2 = xt[:, 2].reshape(G, 128)
    ox, oy, oz, nx, ny, nz, mk = _run(x2, y2, z2)
    out_xt = jnp.stack(
        [ox.reshape(-1), oy.reshape(-1), oz.reshape(-1)], axis=1
    )
    nb = jnp.stack(
        [nx.reshape(-1), ny.reshape(-1), nz.reshape(-1)], axis=1
    )
    return out_xt, nb, mk.reshape(-1).astype(bool)


def kernel(xt):
    return _full(xt)
